# baseline (speedup 1.0000x reference)
import functools

import jax
import jax.numpy as jnp
from jax import lax
from jax.experimental import pallas as pl
from jax.experimental.pallas import tpu as pltpu
from jax.experimental.pallas import tpu_sc as plsc

N = 50000
E = 400000
D = 128
L = 3
ET = 2

NS = 16
CW = 32
NK = D // CW
B = 200
EPT = E // NS
NB = EPT // B
CPB = 5
NCH = NB // CPB
NBR = 2 * E // B
NP = 50176
RPT = NP // NS
DR = 448
ND = RPT // DR

_MESH = dict(
    mesh=plsc.VectorSubcoreMesh(core_axis_name="c", subcore_axis_name="s"),
    compiler_params=pltpu.CompilerParams(use_tc_tiling_on_sc=False),
)


def _make_msum(with_deg):
    out_type = [jax.ShapeDtypeStruct((ET, NP, D), jnp.float32)]
    if with_deg:
        out_type = [jax.ShapeDtypeStruct((ET, NP, D), jnp.float32)] + out_type

    def body(hv, epk, zz, oo, *rest):
        if with_deg:
            dg, mo = rest[0], rest[1]
            scr = rest[2:]
        else:
            mo = rest[0]
            scr = rest[1:]
        acc, srcb, dstb, rows4, gsem, ssem, isem, zsem, z2sem = scr
        c = lax.axis_index("c")
        s = lax.axis_index("s")
        slab0 = s * RPT
        gb0 = c * (E // B) + s * NB

        def zero_acc():
            for j in range(ND):
                pltpu.async_copy(zz, acc.at[pl.ds(slab0 + j * DR, DR)], zsem)
            for j in range(ND):
                pltpu.make_async_copy(zz, acc.at[pl.ds(slab0 + j * DR, DR)], zsem).wait()

        def drain_and_zero(out_ref, k, do_zero):
            col = pl.ds(k * CW, CW)
            for j in range(ND):
                r0 = slab0 + j * DR
                pltpu.async_copy(acc.at[pl.ds(r0, DR)],
                                 out_ref.at[c, pl.ds(r0, DR), col], zsem)
            for j in range(ND):
                r0 = slab0 + j * DR
                pltpu.make_async_copy(acc.at[pl.ds(r0, DR)],
                                      out_ref.at[c, pl.ds(r0, DR), col], zsem).wait()
                if do_zero:
                    pltpu.async_copy(zz, acc.at[pl.ds(r0, DR)], z2sem)
            if do_zero:
                for j in range(ND):
                    pltpu.make_async_copy(
                        zz, acc.at[pl.ds(slab0 + j * DR, DR)], z2sem).wait()

        def edge_loop(k):
            gather = k is not None
            nidx = 2 if gather else 1

            tbl = hv.at[pl.ds(k if gather else 0, NK * (NP - 1) + 1)]

            def chunk_load_async(ch, slot):
                row = pl.ds(gb0 + ch * CPB, CPB)
                if gather:
                    pltpu.async_copy(epk.at[0, row], srcb.at[slot], isem)
                pltpu.async_copy(epk.at[1, row], dstb.at[slot], isem)

            def chunk_load_wait():
                for _ in range(nidx):
                    pltpu.make_async_copy(
                        epk.at[1, pl.ds(gb0, CPB)], dstb.at[0], isem).wait()

            def body_fn(j, carry):
                p = j % 4
                sp = j % 2
                q = (j // CPB) % 2
                r = j % CPB
                if gather:
                    pltpu.make_async_copy(
                        tbl.at[srcb.at[q, r]], rows4.at[p], gsem.at[sp]).wait()
                src_slot = rows4.at[p] if gather else rows4.at[0]

                @pl.when(j >= 2)
                def _():
                    pltpu.make_async_copy(
                        rows4.at[(j + 2) % 4] if gather else rows4.at[0],
                        acc.at[dstb.at[q, r]], ssem.at[sp]).wait()

                ch1 = j // CPB + 1
                @pl.when((r == 2) & (ch1 < NCH))
                def _():
                    chunk_load_async(ch1, 1 - q)

                if gather:
                    @pl.when(j + 2 < NB)
                    def _():
                        j2 = j + 2
                        q2 = (j2 // CPB) % 2
                        r2 = j2 % CPB
                        @pl.when(r2 == 0)
                        def _():
                            chunk_load_wait()
                        pltpu.async_copy(
                            tbl.at[srcb.at[q2, r2]], rows4.at[j2 % 4], gsem.at[sp])
                else:
                    @pl.when((r == CPB - 1) & (j + 1 < NB))
                    def _():
                        chunk_load_wait()

                pltpu.async_copy(src_slot, acc.at[dstb.at[q, r]], ssem.at[sp],
                                 add=True)
                return carry

            chunk_load_async(0, 0)
            chunk_load_wait()
            if gather:
                pltpu.async_copy(tbl.at[srcb.at[0, 0]], rows4.at[0], gsem.at[0])
                pltpu.async_copy(tbl.at[srcb.at[0, 1]], rows4.at[1], gsem.at[1])
            lax.fori_loop(0, NB, body_fn, 0)
            pltpu.make_async_copy(
                rows4.at[(NB - 2) % 4] if gather else rows4.at[0],
                acc.at[dstb.at[0, 0]], ssem.at[(NB - 2) % 2]).wait()
            pltpu.make_async_copy(
                rows4.at[(NB - 1) % 4] if gather else rows4.at[0],
                acc.at[dstb.at[0, 0]], ssem.at[(NB - 1) % 2]).wait()

        zero_acc()
        if with_deg:
            pltpu.sync_copy(oo, rows4.at[0])
        plsc.subcore_barrier()
        if with_deg:
            edge_loop(None)
            plsc.subcore_barrier()
            drain_and_zero(dg, 0, True)
            plsc.subcore_barrier()
        for k in range(NK):
            edge_loop(k)
            plsc.subcore_barrier()
            drain_and_zero(mo, k, k < NK - 1)
            if k < NK - 1:
                plsc.subcore_barrier()

    return pl.kernel(
        body,
        out_type=out_type,
        scratch_types=[
            pltpu.VMEM_SHARED((NP, CW), jnp.float32),
            pltpu.VMEM((2, CPB, B), jnp.int32),
            pltpu.VMEM((2, CPB, B), jnp.int32),
            pltpu.VMEM((4, B, CW), jnp.float32),
            pltpu.SemaphoreType.DMA((2,)),
            pltpu.SemaphoreType.DMA((2,)),
            pltpu.SemaphoreType.DMA,
            pltpu.SemaphoreType.DMA,
            pltpu.SemaphoreType.DMA,
        ],
        **_MESH,
    )


_msum0 = _make_msum(with_deg=True)
_msum = _make_msum(with_deg=False)


BNS = 1568
BNF = 1000


def _self_body(h_ref, ws_ref, b_ref, out_ref):
    h = h_ref[...]
    for e in range(ET):
        out_ref[e] = (
            jnp.dot(h, ws_ref[e], preferred_element_type=jnp.float32) + b_ref[e]
        )


def _dense_self(h, ws, bb):
    return pl.pallas_call(
        _self_body,
        grid=(NP // BNS,),
        in_specs=[
            pl.BlockSpec((BNS, D), lambda i: (i, 0)),
            pl.BlockSpec((ET, D, D), lambda i: (0, 0, 0)),
            pl.BlockSpec((ET, D), lambda i: (0, 0)),
        ],
        out_specs=pl.BlockSpec((ET, BNS, D), lambda i: (0, i, 0)),
        out_shape=jax.ShapeDtypeStruct((ET, NP, D), jnp.float32),
    )(h, ws, bb)


def _combine_body(s_ref, m_ref, deg_ref, wn_ref, out_ref, *, final, bn):
    out = jnp.zeros((bn, D), jnp.float32)
    for e in range(ET):
        inv = 1.0 / jnp.maximum(deg_ref[e], 1.0)
        he = s_ref[e] + jnp.dot(m_ref[e] * inv, wn_ref[e],
                                preferred_element_type=jnp.float32)
        if not final:
            he = jnp.maximum(he, 0.0)
        out = out + he
    out_ref[...] = out


def _dense_combine(s, ms, deg, wn, *, final):
    bn = BNF if final else BNS
    rows = N if final else NP
    return pl.pallas_call(
        functools.partial(_combine_body, final=final, bn=bn),
        grid=(rows // bn,),
        in_specs=[
            pl.BlockSpec((ET, bn, D), lambda i: (0, i, 0)),
            pl.BlockSpec((ET, bn, D), lambda i: (0, i, 0)),
            pl.BlockSpec((ET, bn, 1), lambda i: (0, i, 0)),
            pl.BlockSpec((ET, D, D), lambda i: (0, 0, 0)),
        ],
        out_specs=pl.BlockSpec((bn, D), lambda i: (i, 0)),
        out_shape=jax.ShapeDtypeStruct((rows, D), jnp.float32),
    )(s, ms, deg, wn)


def kernel(x, edge_index_0, edge_index_1, W_self, W_neigh, b):
    se = jnp.concatenate([edge_index_0[0], edge_index_1[0]]).reshape(NBR, B)
    de = jnp.concatenate([edge_index_0[1], edge_index_1[1]]).reshape(NBR, B)
    epk = jnp.stack([se * NK, de])
    zz = jnp.zeros((DR, CW), jnp.float32)
    oo = jnp.ones((B, CW), jnp.float32)
    h = jnp.pad(x, ((0, NP - N), (0, 0)))
    deg = None
    for l in range(L):
        final = l == L - 1
        hv = h.reshape(NP * NK, CW)
        if l == 0:
            dg, ms = _msum0(hv, epk, zz, oo)
            deg = dg[:, :, 0:1]
        else:
            (ms,) = _msum(hv, epk, zz, oo)
        s = _dense_self(h, W_self[l], b[l])
        h = _dense_combine(s, ms, deg, W_neigh[l], final=final)
    return h
